# baseline (speedup 1.0000x reference)
import functools

import jax
import jax.numpy as jnp
from jax import lax
from jax.experimental import pallas as pl
from jax.experimental.pallas import tpu as pltpu
from jax.experimental.pallas import tpu_sc as plsc

NU = 5000
N = 10000
D = 128
E = 320000
NC = 2
NS = 16
RPB = 128
EP = 327680
RP = EP // RPB
R_T32 = RP // (NC * NS)
NPAD = 10240
RT = NPAD // NS

_sc_mesh = plsc.VectorSubcoreMesh(core_axis_name="c", subcore_axis_name="s")


@functools.partial(
    pl.kernel,
    out_type=jax.ShapeDtypeStruct((NC * NPAD, D), jnp.float32),
    mesh=_sc_mesh,
    scratch_types=(
        pltpu.VMEM((RPB,), jnp.int32),
        pltpu.VMEM((RPB,), jnp.int32),
        pltpu.VMEM((RPB,), jnp.int32),
        pltpu.VMEM((RPB,), jnp.int32),
        pltpu.VMEM((RPB,), jnp.int32),
        pltpu.VMEM((RPB,), jnp.int32),
        pltpu.VMEM((RPB,), jnp.int32),
        pltpu.VMEM((RPB,), jnp.int32),
        pltpu.VMEM((RPB, D), jnp.float32),
        pltpu.VMEM((RPB, D), jnp.float32),
        pltpu.VMEM_SHARED((NPAD, D), jnp.float32),
        pltpu.SemaphoreType.DMA,
        pltpu.SemaphoreType.DMA,
        pltpu.SemaphoreType.DMA,
        pltpu.SemaphoreType.DMA,
        pltpu.SemaphoreType.DMA,
        pltpu.SemaphoreType.DMA,
        pltpu.SemaphoreType.DMA,
        pltpu.SemaphoreType.DMA,
        pltpu.SemaphoreType.DMA,
        pltpu.SemaphoreType.DMA,
    ),
)
def _sc_gcn(t_hbm, src_hbm, dst_hbm, zblk_hbm, out,
            sidx0, sidx1, sidx2, sidx3, didx0, didx1, didx2, didx3,
            rows0, rows1, acc,
            ss0, ss1, ss2, ss3, ds0, ds1, ds2, ds3, g0, g1):
    c = lax.axis_index("c")
    s = lax.axis_index("s")
    wid = s * NC + c
    base = wid * R_T32 * RPB
    sidx = (sidx0, sidx1, sidx2, sidx3)
    didx = (didx0, didx1, didx2, didx3)
    rows = (rows0, rows1)
    ssem = (ss0, ss1, ss2, ss3)
    dsem = (ds0, ds1, ds2, ds3)
    gsem = (g0, g1)
    pltpu.sync_copy(zblk_hbm, acc.at[pl.ds(s * RT, RT)])
    plsc.subcore_barrier()

    def issue_idx(koff, q):
        pltpu.async_copy(src_hbm.at[pl.ds(base + koff, RPB)], sidx[q], ssem[q])
        pltpu.async_copy(dst_hbm.at[pl.ds(base + koff, RPB)], didx[q], dsem[q])

    def issue_gather(q, p):
        pltpu.async_copy(t_hbm.at[sidx[q]], rows[p], gsem[p])

    for i in range(4):
        issue_idx(i * RPB, i)
    for i in range(2):
        pltpu.make_async_copy(src_hbm.at[pl.ds(base + i * RPB, RPB)],
                              sidx[i], ssem[i]).wait()
        issue_gather(i, i)

    def step(koff, k_static):
        p = k_static % 2
        q = k_static % 4
        q2 = (k_static + 2) % 4
        pltpu.make_async_copy(t_hbm.at[sidx[q]], rows[p], gsem[p]).wait()
        pltpu.make_async_copy(dst_hbm.at[pl.ds(base + koff, RPB)],
                              didx[q], dsem[q]).wait()
        pltpu.sync_copy(rows[p], acc.at[didx[q]], add=True)
        issue_idx(koff + 4 * RPB, q)
        pltpu.make_async_copy(src_hbm.at[pl.ds(base + koff + 2 * RPB, RPB)],
                              sidx[q2], ssem[q2]).wait()
        issue_gather(q2, p)

    def body(j, carry):
        b0 = j * (4 * RPB)
        for i in range(4):
            step(b0 + i * RPB, i)
        return carry

    NBLK = (R_T32 - 4) // 4
    lax.fori_loop(0, NBLK, body, 0)

    eb = (R_T32 - 4) * RPB
    for i in range(2):
        k = R_T32 - 4 + i
        p, q, q2 = k % 2, k % 4, (k + 2) % 4
        pltpu.make_async_copy(t_hbm.at[sidx[q]], rows[p], gsem[p]).wait()
        pltpu.make_async_copy(dst_hbm.at[pl.ds(eb + i * RPB, RPB)],
                              didx[q], dsem[q]).wait()
        pltpu.sync_copy(rows[p], acc.at[didx[q]], add=True)
        pltpu.make_async_copy(src_hbm.at[pl.ds(eb + (i + 2) * RPB, RPB)],
                              sidx[q2], ssem[q2]).wait()
        issue_gather(q2, p)
    for i in range(2, 4):
        k = R_T32 - 4 + i
        p, q = k % 2, k % 4
        pltpu.make_async_copy(t_hbm.at[sidx[q]], rows[p], gsem[p]).wait()
        pltpu.make_async_copy(dst_hbm.at[pl.ds(eb + i * RPB, RPB)],
                              didx[q], dsem[q]).wait()
        pltpu.sync_copy(rows[p], acc.at[didx[q]], add=True)

    plsc.subcore_barrier()
    pltpu.sync_copy(acc.at[pl.ds(s * RT, RT)],
                    out.at[pl.ds(c * NPAD + s * RT, RT)])


_BLK1 = 128


def _tc_xform_body(x_ref, wut_ref, wit_ref, bu_ref, bi_ref, oi_ref,
                   node_ref, t_ref):
    i = pl.program_id(0)
    x = x_ref[...]
    yu = jnp.dot(x, wut_ref[...], preferred_element_type=jnp.float32)
    yi = jnp.dot(x, wit_ref[...], preferred_element_type=jnp.float32)
    row = i * _BLK1 + lax.broadcasted_iota(jnp.int32, (_BLK1, 1), 0)
    y = jnp.where(row < NU, yu + bu_ref[...], yi + bi_ref[...])
    n = jnp.maximum(y, 0.0)
    node_ref[...] = n
    t_ref[...] = n * oi_ref[...]


def _tc_xform(x_pad, wut, wit, bu, bi, oi_col):
    return pl.pallas_call(
        _tc_xform_body,
        grid=(NPAD // _BLK1,),
        in_specs=[
            pl.BlockSpec((_BLK1, D), lambda i: (i, 0)),
            pl.BlockSpec((D, D), lambda i: (0, 0)),
            pl.BlockSpec((D, D), lambda i: (0, 0)),
            pl.BlockSpec((1, D), lambda i: (0, 0)),
            pl.BlockSpec((1, D), lambda i: (0, 0)),
            pl.BlockSpec((_BLK1, 1), lambda i: (i, 0)),
        ],
        out_specs=[
            pl.BlockSpec((_BLK1, D), lambda i: (i, 0)),
            pl.BlockSpec((_BLK1, D), lambda i: (i, 0)),
        ],
        out_shape=[
            jax.ShapeDtypeStruct((NPAD, D), jnp.float32),
            jax.ShapeDtypeStruct((NPAD, D), jnp.float32),
        ],
    )(x_pad, wut, wit, bu, bi, oi_col)


_BLK2 = 1280


def _tc_mid_body(pa_ref, pb_ref, ii_ref, oi_ref, emb_ref, t_ref):
    e = (pa_ref[0] + pb_ref[0]) * ii_ref[...]
    emb_ref[...] = e
    t_ref[...] = e * oi_ref[...]


def _tc_mid(p3d, ii_col, oi_col):
    return pl.pallas_call(
        _tc_mid_body,
        grid=(NPAD // _BLK2,),
        in_specs=[
            pl.BlockSpec((1, _BLK2, D), lambda i: (0, i, 0)),
            pl.BlockSpec((1, _BLK2, D), lambda i: (1, i, 0)),
            pl.BlockSpec((_BLK2, 1), lambda i: (i, 0)),
            pl.BlockSpec((_BLK2, 1), lambda i: (i, 0)),
        ],
        out_specs=[
            pl.BlockSpec((_BLK2, D), lambda i: (i, 0)),
            pl.BlockSpec((_BLK2, D), lambda i: (i, 0)),
        ],
        out_shape=[
            jax.ShapeDtypeStruct((NPAD, D), jnp.float32),
            jax.ShapeDtypeStruct((NPAD, D), jnp.float32),
        ],
    )(p3d, p3d, ii_col, oi_col)


def _tc_out_user_body(node_ref, emb0_ref, pa_ref, pb_ref, ii_ref, o_ref):
    emb1 = (pa_ref[0] + pb_ref[0]) * ii_ref[...]
    o_ref[...] = node_ref[...] + emb0_ref[...] * 0.5 + emb1 * (1.0 / 3.0)


def _tc_out_item_body(node_ref, emb0_ref, pa_ref, pb_ref, ii_ref, side_ref,
                      o_ref):
    emb1 = (pa_ref[0] + pb_ref[0]) * ii_ref[...]
    o_ref[...] = (node_ref[...] + emb0_ref[...] * 0.5 + emb1 * (1.0 / 3.0)
                  + side_ref[...])


def _tc_out(node0, emb0, p3d, ii_col, side):
    user = pl.pallas_call(
        _tc_out_user_body,
        grid=(1,),
        in_specs=[
            pl.BlockSpec((NU, D), lambda i: (0, 0)),
            pl.BlockSpec((NU, D), lambda i: (0, 0)),
            pl.BlockSpec((1, NU, D), lambda i: (0, 0, 0)),
            pl.BlockSpec((1, NU, D), lambda i: (1, 0, 0)),
            pl.BlockSpec((NU, 1), lambda i: (0, 0)),
        ],
        out_specs=pl.BlockSpec((NU, D), lambda i: (0, 0)),
        out_shape=jax.ShapeDtypeStruct((NU, D), jnp.float32),
    )(node0, emb0, p3d, p3d, ii_col)
    item = pl.pallas_call(
        _tc_out_item_body,
        grid=(1,),
        in_specs=[
            pl.BlockSpec((NU, D), lambda i: (1, 0)),
            pl.BlockSpec((NU, D), lambda i: (1, 0)),
            pl.BlockSpec((1, NU, D), lambda i: (0, 1, 0)),
            pl.BlockSpec((1, NU, D), lambda i: (1, 1, 0)),
            pl.BlockSpec((NU, 1), lambda i: (1, 0)),
            pl.BlockSpec((NU, D), lambda i: (0, 0)),
        ],
        out_specs=pl.BlockSpec((NU, D), lambda i: (0, 0)),
        out_shape=jax.ShapeDtypeStruct((NU, D), jnp.float32),
    )(node0, emb0, p3d, p3d, ii_col, side)
    return user, item


def kernel(origin_user_embedding, origin_item_embedding, item_side_feat,
           Wu, bu, Wi, bi, edge_index):
    src = edge_index[0]
    dst = edge_index[1]
    pad = EP - E
    pad_src = (jnp.arange(pad, dtype=jnp.int32) % 128) + N
    pad_dst = (jnp.arange(pad, dtype=jnp.int32) % 112) + N + 128
    srcp = jnp.concatenate([src, pad_src])
    dstp = jnp.concatenate([dst, pad_dst])
    zblk = jnp.zeros((RT, D), jnp.float32)

    out_deg = jnp.maximum(
        jnp.bincount(src, length=N).astype(jnp.float32), 1.0
    )
    in_deg = jnp.maximum(
        jnp.bincount(dst, length=N).astype(jnp.float32), 1.0
    )
    oi_col = jnp.pad(lax.rsqrt(out_deg), (0, NPAD - N),
                     constant_values=1.0)[:, None]
    ii_col = jnp.pad(lax.rsqrt(in_deg), (0, NPAD - N),
                     constant_values=1.0)[:, None]

    x_pad = jnp.concatenate(
        [origin_user_embedding, origin_item_embedding,
         jnp.zeros((NPAD - N, D), jnp.float32)], axis=0)
    node0, t0 = _tc_xform(x_pad, Wu.T, Wi.T, bu[None, :], bi[None, :], oi_col)

    p0 = _sc_gcn(t0, srcp, dstp, zblk).reshape(NC, NPAD, D)
    emb0, t1 = _tc_mid(p0, ii_col, oi_col)

    p1 = _sc_gcn(t1, srcp, dstp, zblk).reshape(NC, NPAD, D)
    return _tc_out(node0, emb0, p1, ii_col, item_side_feat)
